# Initial kernel scaffold; baseline (speedup 1.0000x reference)
import jax
import jax.numpy as jnp
from jax.experimental import pallas as pl


def kernel(x, Wr1, br1, Wr2, br2, Wr3, br3, W1, b1, W2, b2):
    raise NotImplementedError("write your pallas kernel here")


# speedup vs baseline: 4.0924x; 4.0924x over previous
import functools

import jax
import jax.numpy as jnp
import numpy as np
from jax.experimental import pallas as pl

B, T, C, E, K = 2, 2048, 768, 8, 2
CF = 1.25
CAP = int(T / E * CF)
H4 = 4 * C

_SEL_TABLE = np.stack(
    [
        np.linspace(0, n - 1, CAP).astype(np.int64)
        if n > CAP
        else np.zeros(CAP, np.int64)
        for n in range(T + 1)
    ]
).astype(np.float32)
_SEL_PAD = np.zeros((2056, 384), np.float32)
_SEL_PAD[: T + 1, :CAP] = _SEL_TABLE
_SEL_PAD_T = np.ascontiguousarray(_SEL_PAD.T)

_TRI = np.triu(np.ones((T, T), np.float32), 1).astype(jnp.bfloat16)


def _router_body(x_ref, w1_ref, b1_ref, w2_ref, b2_ref, w3_ref, b3_ref,
                 tp_ref, te_ref):
    xb = x_ref[...]
    h = jax.nn.relu(jnp.dot(xb, w1_ref[...],
                            preferred_element_type=jnp.float32) + b1_ref[...])
    h = jax.nn.relu(jnp.dot(h, w2_ref[...],
                            preferred_element_type=jnp.float32) + b2_ref[...])
    logits = jnp.dot(h, w3_ref[...],
                     preferred_element_type=jnp.float32) + b3_ref[...]
    p = jax.nn.softmax(logits, axis=-1)
    m1 = jnp.max(p, axis=-1)
    a1 = jnp.argmax(p, axis=-1).astype(jnp.int32)
    cols = jax.lax.broadcasted_iota(jnp.int32, p.shape, 1)
    pm = jnp.where(cols == a1[:, None], jnp.float32(-1.0), p)
    m2 = jnp.max(pm, axis=-1)
    a2 = jnp.argmax(pm, axis=-1).astype(jnp.int32)
    tp_ref[0, 0, :] = m1
    tp_ref[0, 1, :] = m2
    te_ref[0, 0, :] = a1
    te_ref[0, 1, :] = a2


def _router(x2d, Wr1, br1, Wr2, br2, Wr3, br3):
    TB = 256
    n_tb = (B * T) // TB
    per_b = T // TB
    grid = (n_tb,)
    out_shape = [
        jax.ShapeDtypeStruct((B, K, T), jnp.float32),
        jax.ShapeDtypeStruct((B, K, T), jnp.int32),
    ]
    in_specs = [
        pl.BlockSpec((TB, C), lambda i: (i, 0)),
        pl.BlockSpec((C, H4), lambda i: (0, 0)),
        pl.BlockSpec((1, H4), lambda i: (0, 0)),
        pl.BlockSpec((H4, H4), lambda i: (0, 0)),
        pl.BlockSpec((1, H4), lambda i: (0, 0)),
        pl.BlockSpec((H4, E), lambda i: (0, 0)),
        pl.BlockSpec((1, E), lambda i: (0, 0)),
    ]
    out_specs = [
        pl.BlockSpec((1, K, TB), lambda i: (i // per_b, 0, i % per_b)),
        pl.BlockSpec((1, K, TB), lambda i: (i // per_b, 0, i % per_b)),
    ]
    return pl.pallas_call(
        _router_body, grid=grid, in_specs=in_specs, out_specs=out_specs,
        out_shape=out_shape,
    )(x2d, Wr1, br1.reshape(1, H4), Wr2, br2.reshape(1, H4), Wr3,
      br3.reshape(1, E))


NH = 4
HC = H4 // NH


def _moe_body(x_ref, te_ref, tp_ref, selt_ref, tri_ref, w1_ref, b1_ref,
              w2_ref, b2_ref, y_ref, xg_s, ohw_s):
    e_id = pl.program_id(1)
    h_id = pl.program_id(2)

    @pl.when(h_id == 0)
    def _plan():
        teb = te_ref[0]
        tpb = tp_ref[0]
        m = teb == e_id
        m0 = m[0:1, :]
        m1 = m[1:2, :]
        anym = jnp.logical_or(m0, m1)
        af = anym.astype(jnp.float32)

        rank_row = jnp.dot(af.astype(jnp.bfloat16), tri_ref[...],
                           preferred_element_type=jnp.float32)

        count = jnp.sum(af)
        count_i = count.astype(jnp.int32)

        oh_cnt = (jax.lax.broadcasted_iota(jnp.int32, (1, 2056), 1)
                  == count_i).astype(jnp.float32)
        sel_col = jax.lax.dot_general(
            selt_ref[...], oh_cnt, (((1,), (1,)), ((), ())),
            preferred_element_type=jnp.float32)
        sel_col = sel_col[:CAP, :]
        jcap = jax.lax.broadcasted_iota(jnp.int32, (CAP, 1), 0)
        jcap = jcap.astype(jnp.float32)
        over = count > jnp.float32(CAP)
        sel_eff = jnp.where(over, sel_col, jcap)
        validj = jnp.logical_or(over, jcap < count)

        oht = jnp.logical_and(sel_eff == rank_row, anym)
        oht = jnp.logical_and(oht, validj).astype(jnp.float32)

        xb = x_ref[0]
        xg_s[...] = jnp.dot(oht, xb,
                            preferred_element_type=jnp.float32)
        p_slot = jnp.where(m0, tpb[0:1, :], tpb[1:2, :])
        ohw_s[...] = oht * p_slot

    hc = jax.nn.relu(jnp.dot(xg_s[...], w1_ref[0],
                             preferred_element_type=jnp.float32) + b1_ref[0])
    og_c = jnp.dot(hc, w2_ref[0],
                   preferred_element_type=jnp.float32)
    og_c = og_c + b2_ref[0] * (h_id == 0).astype(jnp.float32)
    contrib = jax.lax.dot_general(ohw_s[...], og_c, (((0,), (0,)), ((), ())),
                                  preferred_element_type=jnp.float32)

    first = jnp.logical_and(e_id == 0, h_id == 0)

    @pl.when(first)
    def _():
        y_ref[0] = contrib

    @pl.when(jnp.logical_not(first))
    def _():
        y_ref[0] = y_ref[0] + contrib


def _moe(x, te, tp, selt, tri, W1, b1, W2, b2):
    from jax.experimental.pallas import tpu as pltpu
    grid = (B, E, NH)
    in_specs = [
        pl.BlockSpec((1, T, C), lambda b, e, h: (b, 0, 0)),
        pl.BlockSpec((1, K, T), lambda b, e, h: (b, 0, 0)),
        pl.BlockSpec((1, K, T), lambda b, e, h: (b, 0, 0)),
        pl.BlockSpec((384, 2056), lambda b, e, h: (0, 0)),
        pl.BlockSpec((T, T), lambda b, e, h: (0, 0)),
        pl.BlockSpec((1, C, HC), lambda b, e, h: (e, 0, h)),
        pl.BlockSpec((1, 1, HC), lambda b, e, h: (e, 0, h)),
        pl.BlockSpec((1, HC, C), lambda b, e, h: (e, h, 0)),
        pl.BlockSpec((1, 1, C), lambda b, e, h: (e, 0, 0)),
    ]
    out_specs = pl.BlockSpec((1, T, C), lambda b, e, h: (b, 0, 0))
    return pl.pallas_call(
        _moe_body, grid=grid, in_specs=in_specs, out_specs=out_specs,
        out_shape=jax.ShapeDtypeStruct((B, T, C), jnp.float32),
        scratch_shapes=[pltpu.VMEM((CAP, C), jnp.float32),
                        pltpu.VMEM((CAP, T), jnp.float32)],
    )(x, te, tp, selt, tri, W1, b1.reshape(E, 1, H4), W2, b2.reshape(E, 1, C))


@jax.jit
def kernel(x, Wr1, br1, Wr2, br2, Wr3, br3, W1, b1, W2, b2):
    x2d = x.reshape(B * T, C)
    tp, te = _router(x2d, Wr1, br1, Wr2, br2, Wr3, br3)
    selt = jnp.asarray(_SEL_PAD_T)
    tri = jnp.asarray(_TRI)
    return _moe(x, te, tp, selt, tri, W1, b1, W2, b2)
